# baseline (speedup 1.0000x reference)
import functools

import jax
import jax.numpy as jnp
from jax import lax
from jax.experimental import pallas as pl
from jax.experimental.pallas import tpu as pltpu
from jax.experimental.pallas import tpu_sc as plsc

_EPS = 1e-5
_DTC = 0.01


def _ln(y, g, b):
    mu = jnp.mean(y, axis=-1, keepdims=True)
    var = jnp.mean((y - mu) ** 2, axis=-1, keepdims=True)
    return (y - mu) * lax.rsqrt(var + _EPS) * g + b


def _w(shape):
    return pl.BlockSpec(shape, lambda i: tuple(0 for _ in shape))


def _row(block_rows, cols):
    return pl.BlockSpec((block_rows, cols), lambda i: (i, 0))


def _kron4(w):
    return jnp.kron(jnp.eye(4, dtype=jnp.float32), w)


def _t4(v):
    return jnp.tile(v, 4).reshape(1, -1)


def _mavg():
    return jnp.kron(jnp.eye(4, dtype=jnp.float32),
                    jnp.full((32, 32), 1.0 / 32.0, jnp.float32))


def _pln(y, mavg, g, b):
    mu = jnp.dot(y, mavg, preferred_element_type=jnp.float32)
    d = y - mu
    var = jnp.dot(d * d, mavg, preferred_element_type=jnp.float32)
    return d * lax.rsqrt(var + _EPS) * g + b


def _dot(a, b):
    return jnp.dot(a, b, preferred_element_type=jnp.float32)


def _tc_node_encoder(x_p, mp, lnp, wa, wb):
    r = x_p.shape[0]

    def body(x_ref, w1_ref, b1_ref, w2_ref, b2_ref, g_ref, gb_ref, mavg_ref,
             wa_ref, wb_ref, hn_ref, a_ref, b_ref, m_ref):
        xx = x_ref[...]
        h = jnp.maximum(_dot(xx, w1_ref[...]) + b1_ref[...], 0.0)
        y = _dot(h, w2_ref[...]) + b2_ref[...]
        hn = _pln(y, mavg_ref[...], g_ref[...], gb_ref[...])
        hn_ref[...] = hn
        a_ref[...] = _dot(hn, wa_ref[...])
        b_ref[...] = _dot(hn, wb_ref[...])
        cols = []
        for gidx in range(4):
            z0 = xx[:, 128 * gidx + 1:128 * gidx + 2]
            t1 = xx[:, 128 * gidx + 2:128 * gidx + 3] + _DTC
            mg = (z0 <= t1).astype(jnp.float32)
            cols += [mg, mg, mg]
        m_ref[...] = jnp.concatenate(cols, axis=1)

    return pl.pallas_call(
        body,
        grid=(1,),
        in_specs=[_w((r, 512)), _w((512, 128)), _w((1, 128)), _w((128, 128)),
                  _w((1, 128)), _w((1, 128)), _w((1, 128)), _w((128, 128)),
                  _w((128, 128)), _w((128, 128))],
        out_specs=[_w((r, 128)), _w((r, 128)), _w((r, 128)), _w((r, 12))],
        out_shape=[jax.ShapeDtypeStruct((r, 128), jnp.float32),
                   jax.ShapeDtypeStruct((r, 128), jnp.float32),
                   jax.ShapeDtypeStruct((r, 128), jnp.float32),
                   jax.ShapeDtypeStruct((r, 12), jnp.float32)],
    )(x_p, _kron4(mp["w1"]), _t4(mp["b1"]), _kron4(mp["w2"]), _t4(mp["b2"]),
      _t4(lnp["g"]), _t4(lnp["b"]), _mavg(), _kron4(wa), _kron4(wb))


def _tc_edge_encoder(a0, a1, mp, lnp):
    r = a0.shape[0]
    be = 1000
    eye64 = jnp.eye(64, dtype=jnp.float32)
    r0 = jnp.kron(eye64, mp["w1"][0:1])
    r1 = jnp.kron(eye64, mp["w1"][1:2])
    b1big = jnp.tile(mp["b1"], 64).reshape(1, -1)

    def body(a0_ref, a1_ref, r0_ref, r1_ref, b1_ref, w2_ref, b2_ref,
             g_ref, gb_ref, mavg_ref, out_ref):
        h = jnp.maximum(_dot(a0_ref[...], r0_ref[...])
                        + _dot(a1_ref[...], r1_ref[...]) + b1_ref[...], 0.0)
        for t in range(16):
            sl = pl.ds(128 * t, 128)
            y = _dot(h[:, 128 * t:128 * t + 128], w2_ref[...]) + b2_ref[...]
            out_ref[:, sl] = _pln(y, mavg_ref[...], g_ref[...], gb_ref[...])

    return pl.pallas_call(
        body,
        grid=(r // be,),
        in_specs=[_row(be, 64), _row(be, 64), _w((64, 2048)), _w((64, 2048)),
                  _w((1, 2048)), _w((128, 128)), _w((1, 128)), _w((1, 128)),
                  _w((1, 128)), _w((128, 128))],
        out_specs=_row(be, 2048),
        out_shape=jax.ShapeDtypeStruct((r, 2048), jnp.float32),
    )(a0, a1, r0, r1, b1big, _kron4(mp["w2"]), _t4(mp["b2"]), _t4(lnp["g"]),
      _t4(lnp["b"]), _mavg())


def _tc_edge_update(he_p, ga_p, gb_p, mp, lnp):
    r = he_p.shape[0]
    be = 200

    def body(he_ref, ga_ref, gb_ref, w1_ref, b1_ref, w2_ref, b2_ref,
             g_ref, gb2_ref, mavg_ref, out_ref):
        for t in range(16):
            sl = pl.ds(128 * t, 128)
            hh = he_ref[:, sl]
            pre = (_dot(hh, w1_ref[...]) + ga_ref[:, sl] + gb_ref[:, sl]
                   + b1_ref[...])
            h = jnp.maximum(pre, 0.0)
            y = _dot(h, w2_ref[...]) + b2_ref[...]
            out_ref[:, sl] = hh + _pln(y, mavg_ref[...], g_ref[...],
                                       gb2_ref[...])

    return pl.pallas_call(
        body,
        grid=(r // be,),
        in_specs=[_row(be, 2048)] * 3 + [_w((128, 128)), _w((1, 128)),
                                         _w((128, 128)), _w((1, 128)),
                                         _w((1, 128)), _w((1, 128)),
                                         _w((128, 128))],
        out_specs=_row(be, 2048),
        out_shape=jax.ShapeDtypeStruct((r, 2048), jnp.float32),
    )(he_p, ga_p, gb_p, _kron4(mp["w1"][0:32]), _t4(mp["b1"]),
      _kron4(mp["w2"]), _t4(mp["b2"]), _t4(lnp["g"]), _t4(lnp["b"]), _mavg())


def _tc_node_update(hn_p, p0_p, p1_p, mp, lnp, wa=None, wb=None):
    r = hn_p.shape[0]
    emit_ab = wa is not None

    def body(hn_ref, p0_ref, p1_ref, w1a_ref, w1b_ref, b1_ref, w2_ref, b2_ref,
             g_ref, gb_ref, mavg_ref, *rest):
        if emit_ab:
            wa_ref, wb_ref, out_ref, a_ref, b_ref = rest
        else:
            (out_ref,) = rest
        hh = hn_ref[...]
        aggr = p0_ref[...] + p1_ref[...]
        pre = (_dot(hh, w1a_ref[...]) + _dot(aggr, w1b_ref[...]) + b1_ref[...])
        h = jnp.maximum(pre, 0.0)
        y = _dot(h, w2_ref[...]) + b2_ref[...]
        hn_new = hh + _pln(y, mavg_ref[...], g_ref[...], gb_ref[...])
        out_ref[...] = hn_new
        if emit_ab:
            a_ref[...] = _dot(hn_new, wa_ref[...])
            b_ref[...] = _dot(hn_new, wb_ref[...])

    in_specs = [_w((r, 128))] * 3 + [_w((128, 128)), _w((128, 128)),
                                     _w((1, 128)), _w((128, 128)),
                                     _w((1, 128)), _w((1, 128)), _w((1, 128)),
                                     _w((128, 128))]
    args = [hn_p, p0_p, p1_p, _kron4(mp["w1"][0:32]), _kron4(mp["w1"][32:64]),
            _t4(mp["b1"]), _kron4(mp["w2"]), _t4(mp["b2"]),
            _t4(lnp["g"]), _t4(lnp["b"]), _mavg()]
    if emit_ab:
        in_specs += [_w((128, 128)), _w((128, 128))]
        args += [_kron4(wa), _kron4(wb)]
        out_specs = [_w((r, 128))] * 3
        out_shape = [jax.ShapeDtypeStruct((r, 128), jnp.float32)] * 3
    else:
        out_specs = _w((r, 128))
        out_shape = jax.ShapeDtypeStruct((r, 128), jnp.float32)
    return pl.pallas_call(
        body, grid=(1,), in_specs=in_specs, out_specs=out_specs,
        out_shape=out_shape)(*args)


def _tc_decoder(hn_p, m_p, mp):
    r = hn_p.shape[0]

    def body(hn_ref, m_ref, w1_ref, b1_ref, w2_ref, b2_ref, out_ref):
        h = jnp.maximum(_dot(hn_ref[...], w1_ref[...]) + b1_ref[...], 0.0)
        y = _dot(h, w2_ref[...]) + b2_ref[...]
        out_ref[...] = y * m_ref[...]

    return pl.pallas_call(
        body,
        grid=(1,),
        in_specs=[_w((r, 128)), _w((r, 12)), _w((128, 128)), _w((1, 128)),
                  _w((128, 12)), _w((1, 12))],
        out_specs=_w((r, 12)),
        out_shape=jax.ShapeDtypeStruct((r, 12), jnp.float32),
    )(hn_p, m_p, _kron4(mp["w1"]), _t4(mp["b1"]), _kron4(mp["w2"]),
      _t4(mp["b2"]))


@functools.cache
def _mesh():
    return plsc.VectorSubcoreMesh(core_axis_name="c", subcore_axis_name="s")


_NOTILE = pltpu.CompilerParams(use_tc_tiling_on_sc=False)
_CHR = 8
_CH = _CHR * 128


def _sc_gather(a, b, src2, dst2):
    e = src2.shape[0] * 128
    nch = e // _CH
    tail = (e - nch * _CH) // 128

    @functools.partial(
        pl.kernel,
        mesh=_mesh(),
        out_type=[jax.ShapeDtypeStruct((e, 32), jnp.float32),
                  jax.ShapeDtypeStruct((e, 32), jnp.float32)],
        scratch_types=[pltpu.VMEM((_CHR, 128), jnp.int32),
                       pltpu.VMEM((_CHR, 128), jnp.int32),
                       pltpu.VMEM((_CH, 32), jnp.float32),
                       pltpu.VMEM((_CH, 32), jnp.float32),
                       pltpu.SemaphoreType.DMA,
                       pltpu.SemaphoreType.DMA],
        compiler_params=_NOTILE,
    )
    def k(a_hbm, b_hbm, s_hbm, d_hbm, ga_hbm, gb_hbm, si, di, ba, bb, sa, sb):
        c = lax.axis_index("c")
        s = lax.axis_index("s")
        wid = c * 16 + s
        trips = (nch - wid + 31) // 32

        def do_chunk(ch, rows):
            pltpu.sync_copy(s_hbm.at[pl.ds(ch * _CHR, rows)],
                            si.at[pl.ds(0, rows)])
            pltpu.sync_copy(d_hbm.at[pl.ds(ch * _CHR, rows)],
                            di.at[pl.ds(0, rows)])
            cps = []
            for j in range(rows):
                cps.append(pltpu.async_copy(
                    a_hbm.at[si.at[j]], ba.at[pl.ds(j * 128, 128)], sa))
                cps.append(pltpu.async_copy(
                    b_hbm.at[di.at[j]], bb.at[pl.ds(j * 128, 128)], sb))
            for cp in cps:
                cp.wait()
            pltpu.sync_copy(ba.at[pl.ds(0, rows * 128)],
                            ga_hbm.at[pl.ds(ch * _CH, rows * 128)])
            pltpu.sync_copy(bb.at[pl.ds(0, rows * 128)],
                            gb_hbm.at[pl.ds(ch * _CH, rows * 128)])

        def body(i, carry):
            do_chunk(wid + i * 32, _CHR)
            return carry

        lax.fori_loop(0, trips, body, 0)
        if tail:
            @pl.when(wid == 31)
            def _():
                do_chunk(nch, tail)

    return k(a, b, src2, dst2)


def _sc_scatter(he, dst2, zeros):
    n = zeros.shape[0]
    e = he.shape[0]
    nch = e // _CH
    tail = (e - nch * _CH) // 128
    per = n // 16

    @functools.partial(
        pl.kernel,
        mesh=_mesh(),
        out_type=[jax.ShapeDtypeStruct((n, 32), jnp.float32),
                  jax.ShapeDtypeStruct((n, 32), jnp.float32)],
        scratch_types=[pltpu.VMEM((_CHR, 128), jnp.int32),
                       pltpu.VMEM((_CH, 32), jnp.float32),
                       pltpu.VMEM_SHARED((n, 32), jnp.float32)],
        compiler_params=_NOTILE,
    )
    def k(he_hbm, d_hbm, z_hbm, o0, o1, di, be, acc):
        c = lax.axis_index("c")
        s = lax.axis_index("s")
        wid = c * 16 + s
        pltpu.sync_copy(z_hbm.at[pl.ds(s * per, per)], acc.at[pl.ds(s * per, per)])
        plsc.subcore_barrier()
        trips = (nch - wid + 31) // 32

        def do_chunk(ch, rows):
            pltpu.sync_copy(d_hbm.at[pl.ds(ch * _CHR, rows)],
                            di.at[pl.ds(0, rows)])
            pltpu.sync_copy(he_hbm.at[pl.ds(ch * _CH, rows * 128)],
                            be.at[pl.ds(0, rows * 128)])
            for j in range(rows):
                pltpu.sync_copy(be.at[pl.ds(j * 128, 128)], acc.at[di.at[j]],
                                add=True)

        def body(i, carry):
            do_chunk(wid + i * 32, _CHR)
            return carry

        lax.fori_loop(0, trips, body, 0)
        if tail:
            @pl.when(wid == 30)
            def _():
                do_chunk(nch, tail)
        plsc.subcore_barrier()

        @pl.when(c == 0)
        def _():
            pltpu.sync_copy(acc.at[pl.ds(s * per, per)], o0.at[pl.ds(s * per, per)])

        @pl.when(c == 1)
        def _():
            pltpu.sync_copy(acc.at[pl.ds(s * per, per)], o1.at[pl.ds(s * per, per)])

    return k(he, dst2, zeros)


def kernel(x, edge_attr, params, edge_index):
    n = x.shape[0]
    e = edge_index.shape[1]
    src2 = edge_index[0].reshape(-1, 128)
    dst2 = edge_index[1].reshape(-1, 128)
    layers = params["layers"]
    ew = [lp["edge"]["w1"] for lp in layers]

    hn, a, b, m_p = _tc_node_encoder(x.reshape(n // 4, 512), params["enc_n"],
                                     params["enc_n_ln"],
                                     ew[0][32:64], ew[0][64:96])
    he = _tc_edge_encoder(edge_attr[:, 0].reshape(e // 64, 64),
                          edge_attr[:, 1].reshape(e // 64, 64),
                          params["enc_e"], params["enc_e_ln"])
    zeros = jnp.zeros((n, 32), jnp.float32)

    for l, lp in enumerate(layers):
        ga, gb = _sc_gather(a.reshape(n, 32), b.reshape(n, 32), src2, dst2)
        he = _tc_edge_update(he, ga.reshape(e // 64, 2048),
                             gb.reshape(e // 64, 2048), lp["edge"],
                             lp["edge_ln"])
        p0, p1 = _sc_scatter(he.reshape(e, 32), dst2, zeros)
        if l + 1 < len(layers):
            hn, a, b = _tc_node_update(hn, p0.reshape(n // 4, 128),
                                       p1.reshape(n // 4, 128), lp["node"],
                                       lp["node_ln"],
                                       ew[l + 1][32:64], ew[l + 1][64:96])
        else:
            hn = _tc_node_update(hn, p0.reshape(n // 4, 128),
                                 p1.reshape(n // 4, 128), lp["node"],
                                 lp["node_ln"])

    return _tc_decoder(hn, m_p, params["dec"]).reshape(n, 3)


# speedup vs baseline: 1.7718x; 1.7718x over previous
import functools

import jax
import jax.numpy as jnp
from jax import lax
from jax.experimental import pallas as pl
from jax.experimental.pallas import tpu as pltpu
from jax.experimental.pallas import tpu_sc as plsc

_EPS = 1e-5
_DTC = 0.01


def _ln(y, g, b):
    mu = jnp.mean(y, axis=-1, keepdims=True)
    var = jnp.mean((y - mu) ** 2, axis=-1, keepdims=True)
    return (y - mu) * lax.rsqrt(var + _EPS) * g + b


def _w(shape):
    return pl.BlockSpec(shape, lambda i: tuple(0 for _ in shape))


def _row(block_rows, cols):
    return pl.BlockSpec((block_rows, cols), lambda i: (i, 0))


def _kron4(w):
    return jnp.kron(jnp.eye(4, dtype=jnp.float32), w)


def _t4(v):
    return jnp.tile(v, 4).reshape(1, -1)


def _mavg():
    return jnp.kron(jnp.eye(4, dtype=jnp.float32),
                    jnp.full((32, 32), 1.0 / 32.0, jnp.float32))


def _pln(y, mavg, g, b):
    mu = jnp.dot(y, mavg, preferred_element_type=jnp.float32)
    d = y - mu
    var = jnp.dot(d * d, mavg, preferred_element_type=jnp.float32)
    return d * lax.rsqrt(var + _EPS) * g + b


def _dot(a, b):
    return jnp.dot(a, b, preferred_element_type=jnp.float32)


def _tc_node_encoder(x_p, mp, lnp, wa, wb):
    r = x_p.shape[0]

    def body(x_ref, w1_ref, b1_ref, w2_ref, b2_ref, g_ref, gb_ref, mavg_ref,
             wa_ref, wb_ref, hn_ref, a_ref, b_ref, m_ref):
        xx = x_ref[...]
        h = jnp.maximum(_dot(xx, w1_ref[...]) + b1_ref[...], 0.0)
        y = _dot(h, w2_ref[...]) + b2_ref[...]
        hn = _pln(y, mavg_ref[...], g_ref[...], gb_ref[...])
        hn_ref[...] = hn
        a_ref[...] = _dot(hn, wa_ref[...])
        b_ref[...] = _dot(hn, wb_ref[...])
        cols = []
        for gidx in range(4):
            z0 = xx[:, 128 * gidx + 1:128 * gidx + 2]
            t1 = xx[:, 128 * gidx + 2:128 * gidx + 3] + _DTC
            mg = (z0 <= t1).astype(jnp.float32)
            cols += [mg, mg, mg]
        m_ref[...] = jnp.concatenate(cols, axis=1)

    return pl.pallas_call(
        body,
        grid=(1,),
        in_specs=[_w((r, 512)), _w((512, 128)), _w((1, 128)), _w((128, 128)),
                  _w((1, 128)), _w((1, 128)), _w((1, 128)), _w((128, 128)),
                  _w((128, 128)), _w((128, 128))],
        out_specs=[_w((r, 128)), _w((r, 128)), _w((r, 128)), _w((r, 12))],
        out_shape=[jax.ShapeDtypeStruct((r, 128), jnp.float32),
                   jax.ShapeDtypeStruct((r, 128), jnp.float32),
                   jax.ShapeDtypeStruct((r, 128), jnp.float32),
                   jax.ShapeDtypeStruct((r, 12), jnp.float32)],
    )(x_p, _kron4(mp["w1"]), _t4(mp["b1"]), _kron4(mp["w2"]), _t4(mp["b2"]),
      _t4(lnp["g"]), _t4(lnp["b"]), _mavg(), _kron4(wa), _kron4(wb))


def _tc_edge_encoder(a0, a1, mp, lnp):
    r = a0.shape[0]
    be = 1000
    eye64 = jnp.eye(64, dtype=jnp.float32)
    r0 = jnp.kron(eye64, mp["w1"][0:1])
    r1 = jnp.kron(eye64, mp["w1"][1:2])
    b1big = jnp.tile(mp["b1"], 64).reshape(1, -1)

    def body(a0_ref, a1_ref, r0_ref, r1_ref, b1_ref, w2_ref, b2_ref,
             g_ref, gb_ref, mavg_ref, out_ref):
        h = jnp.maximum(_dot(a0_ref[...], r0_ref[...])
                        + _dot(a1_ref[...], r1_ref[...]) + b1_ref[...], 0.0)
        ys = []
        for t in range(16):
            y = _dot(h[:, 128 * t:128 * t + 128], w2_ref[...]) + b2_ref[...]
            ys.append(_pln(y, mavg_ref[...], g_ref[...], gb_ref[...]))
        out_ref[...] = jnp.concatenate(ys, axis=1).reshape(16 * be, 128)

    return pl.pallas_call(
        body,
        grid=(r // be,),
        in_specs=[_row(be, 64), _row(be, 64), _w((64, 2048)), _w((64, 2048)),
                  _w((1, 2048)), _w((128, 128)), _w((1, 128)), _w((1, 128)),
                  _w((1, 128)), _w((128, 128))],
        out_specs=_row(16 * be, 128),
        out_shape=jax.ShapeDtypeStruct((16 * r, 128), jnp.float32),
    )(a0, a1, r0, r1, b1big, _kron4(mp["w2"]), _t4(mp["b2"]), _t4(lnp["g"]),
      _t4(lnp["b"]), _mavg())


def _tc_edge_update(he_p, ga_p, gb_p, mp, lnp):
    r = he_p.shape[0]
    be = 2000

    def body(he_ref, ga_ref, gb_ref, w1_ref, b1_ref, w2_ref, b2_ref,
             g_ref, gb2_ref, mavg_ref, out_ref):
        hh = he_ref[...]
        pre = _dot(hh, w1_ref[...]) + ga_ref[...] + gb_ref[...] + b1_ref[...]
        h = jnp.maximum(pre, 0.0)
        y = _dot(h, w2_ref[...]) + b2_ref[...]
        out_ref[...] = hh + _pln(y, mavg_ref[...], g_ref[...], gb2_ref[...])

    return pl.pallas_call(
        body,
        grid=(r // be,),
        in_specs=[_row(be, 128)] * 3 + [_w((128, 128)), _w((1, 128)),
                                        _w((128, 128)), _w((1, 128)),
                                        _w((1, 128)), _w((1, 128)),
                                        _w((128, 128))],
        out_specs=_row(be, 128),
        out_shape=jax.ShapeDtypeStruct((r, 128), jnp.float32),
    )(he_p, ga_p, gb_p, _kron4(mp["w1"][0:32]), _t4(mp["b1"]),
      _kron4(mp["w2"]), _t4(mp["b2"]), _t4(lnp["g"]), _t4(lnp["b"]), _mavg())


def _tc_node_update(hn_p, p0_p, p1_p, mp, lnp, wa=None, wb=None):
    r = hn_p.shape[0]
    emit_ab = wa is not None

    def body(hn_ref, p0_ref, p1_ref, w1a_ref, w1b_ref, b1_ref, w2_ref, b2_ref,
             g_ref, gb_ref, mavg_ref, *rest):
        if emit_ab:
            wa_ref, wb_ref, out_ref, a_ref, b_ref = rest
        else:
            (out_ref,) = rest
        hh = hn_ref[...]
        aggr = p0_ref[...] + p1_ref[...]
        pre = (_dot(hh, w1a_ref[...]) + _dot(aggr, w1b_ref[...]) + b1_ref[...])
        h = jnp.maximum(pre, 0.0)
        y = _dot(h, w2_ref[...]) + b2_ref[...]
        hn_new = hh + _pln(y, mavg_ref[...], g_ref[...], gb_ref[...])
        out_ref[...] = hn_new
        if emit_ab:
            a_ref[...] = _dot(hn_new, wa_ref[...])
            b_ref[...] = _dot(hn_new, wb_ref[...])

    in_specs = [_w((r, 128))] * 3 + [_w((128, 128)), _w((128, 128)),
                                     _w((1, 128)), _w((128, 128)),
                                     _w((1, 128)), _w((1, 128)), _w((1, 128)),
                                     _w((128, 128))]
    args = [hn_p, p0_p, p1_p, _kron4(mp["w1"][0:32]), _kron4(mp["w1"][32:64]),
            _t4(mp["b1"]), _kron4(mp["w2"]), _t4(mp["b2"]),
            _t4(lnp["g"]), _t4(lnp["b"]), _mavg()]
    if emit_ab:
        in_specs += [_w((128, 128)), _w((128, 128))]
        args += [_kron4(wa), _kron4(wb)]
        out_specs = [_w((r, 128))] * 3
        out_shape = [jax.ShapeDtypeStruct((r, 128), jnp.float32)] * 3
    else:
        out_specs = _w((r, 128))
        out_shape = jax.ShapeDtypeStruct((r, 128), jnp.float32)
    return pl.pallas_call(
        body, grid=(1,), in_specs=in_specs, out_specs=out_specs,
        out_shape=out_shape)(*args)


def _tc_decoder(hn_p, m_p, mp):
    r = hn_p.shape[0]

    def body(hn_ref, m_ref, w1_ref, b1_ref, w2_ref, b2_ref, out_ref):
        h = jnp.maximum(_dot(hn_ref[...], w1_ref[...]) + b1_ref[...], 0.0)
        y = _dot(h, w2_ref[...]) + b2_ref[...]
        out_ref[...] = y * m_ref[...]

    return pl.pallas_call(
        body,
        grid=(1,),
        in_specs=[_w((r, 128)), _w((r, 12)), _w((128, 128)), _w((1, 128)),
                  _w((128, 12)), _w((1, 12))],
        out_specs=_w((r, 12)),
        out_shape=jax.ShapeDtypeStruct((r, 12), jnp.float32),
    )(hn_p, m_p, _kron4(mp["w1"]), _t4(mp["b1"]), _kron4(mp["w2"]),
      _t4(mp["b2"]))


@functools.cache
def _mesh():
    return plsc.VectorSubcoreMesh(core_axis_name="c", subcore_axis_name="s")


_NOTILE = pltpu.CompilerParams(use_tc_tiling_on_sc=False)
_CHR = 8
_CH = _CHR * 128


def _sc_gather(a, b, src2, dst2):
    e = src2.shape[0] * 128
    nch = e // _CH
    tail = (e - nch * _CH) // 128

    @functools.partial(
        pl.kernel,
        mesh=_mesh(),
        out_type=[jax.ShapeDtypeStruct((e, 32), jnp.float32),
                  jax.ShapeDtypeStruct((e, 32), jnp.float32)],
        scratch_types=[pltpu.VMEM((_CHR, 128), jnp.int32),
                       pltpu.VMEM((_CHR, 128), jnp.int32),
                       pltpu.VMEM((_CH, 32), jnp.float32),
                       pltpu.VMEM((_CH, 32), jnp.float32),
                       pltpu.SemaphoreType.DMA,
                       pltpu.SemaphoreType.DMA],
        compiler_params=_NOTILE,
    )
    def k(a_hbm, b_hbm, s_hbm, d_hbm, ga_hbm, gb_hbm, si, di, ba, bb, sa, sb):
        c = lax.axis_index("c")
        s = lax.axis_index("s")
        wid = c * 16 + s
        trips = (nch - wid + 31) // 32

        def do_chunk(ch, rows):
            pltpu.sync_copy(s_hbm.at[pl.ds(ch * _CHR, rows)],
                            si.at[pl.ds(0, rows)])
            pltpu.sync_copy(d_hbm.at[pl.ds(ch * _CHR, rows)],
                            di.at[pl.ds(0, rows)])
            cps = []
            for j in range(rows):
                cps.append(pltpu.async_copy(
                    a_hbm.at[si.at[j]], ba.at[pl.ds(j * 128, 128)], sa))
                cps.append(pltpu.async_copy(
                    b_hbm.at[di.at[j]], bb.at[pl.ds(j * 128, 128)], sb))
            for cp in cps:
                cp.wait()
            pltpu.sync_copy(ba.at[pl.ds(0, rows * 128)],
                            ga_hbm.at[pl.ds(ch * _CH, rows * 128)])
            pltpu.sync_copy(bb.at[pl.ds(0, rows * 128)],
                            gb_hbm.at[pl.ds(ch * _CH, rows * 128)])

        def body(i, carry):
            do_chunk(wid + i * 32, _CHR)
            return carry

        lax.fori_loop(0, trips, body, 0)
        if tail:
            @pl.when(wid == 31)
            def _():
                do_chunk(nch, tail)

    return k(a, b, src2, dst2)


def _sc_scatter(he, dst2, zeros):
    n = zeros.shape[0]
    e = he.shape[0]
    nch = e // _CH
    tail = (e - nch * _CH) // 128
    per = n // 16

    @functools.partial(
        pl.kernel,
        mesh=_mesh(),
        out_type=[jax.ShapeDtypeStruct((n, 32), jnp.float32),
                  jax.ShapeDtypeStruct((n, 32), jnp.float32)],
        scratch_types=[pltpu.VMEM((_CHR, 128), jnp.int32),
                       pltpu.VMEM((_CH, 32), jnp.float32),
                       pltpu.VMEM_SHARED((n, 32), jnp.float32)],
        compiler_params=_NOTILE,
    )
    def k(he_hbm, d_hbm, z_hbm, o0, o1, di, be, acc):
        c = lax.axis_index("c")
        s = lax.axis_index("s")
        wid = c * 16 + s
        pltpu.sync_copy(z_hbm.at[pl.ds(s * per, per)], acc.at[pl.ds(s * per, per)])
        plsc.subcore_barrier()
        trips = (nch - wid + 31) // 32

        def do_chunk(ch, rows):
            pltpu.sync_copy(d_hbm.at[pl.ds(ch * _CHR, rows)],
                            di.at[pl.ds(0, rows)])
            pltpu.sync_copy(he_hbm.at[pl.ds(ch * _CH, rows * 128)],
                            be.at[pl.ds(0, rows * 128)])
            for j in range(rows):
                pltpu.sync_copy(be.at[pl.ds(j * 128, 128)], acc.at[di.at[j]],
                                add=True)

        def body(i, carry):
            do_chunk(wid + i * 32, _CHR)
            return carry

        lax.fori_loop(0, trips, body, 0)
        if tail:
            @pl.when(wid == 30)
            def _():
                do_chunk(nch, tail)
        plsc.subcore_barrier()

        @pl.when(c == 0)
        def _():
            pltpu.sync_copy(acc.at[pl.ds(s * per, per)], o0.at[pl.ds(s * per, per)])

        @pl.when(c == 1)
        def _():
            pltpu.sync_copy(acc.at[pl.ds(s * per, per)], o1.at[pl.ds(s * per, per)])

    return k(he, dst2, zeros)


def kernel(x, edge_attr, params, edge_index):
    n = x.shape[0]
    e = edge_index.shape[1]
    src2 = edge_index[0].reshape(-1, 128)
    dst2 = edge_index[1].reshape(-1, 128)
    layers = params["layers"]
    ew = [lp["edge"]["w1"] for lp in layers]

    hn, a, b, m_p = _tc_node_encoder(x.reshape(n // 4, 512), params["enc_n"],
                                     params["enc_n_ln"],
                                     ew[0][32:64], ew[0][64:96])
    he = _tc_edge_encoder(edge_attr[:, 0].reshape(e // 64, 64),
                          edge_attr[:, 1].reshape(e // 64, 64),
                          params["enc_e"], params["enc_e_ln"])
    zeros = jnp.zeros((n, 32), jnp.float32)

    for l, lp in enumerate(layers):
        ga, gb = _sc_gather(a.reshape(n, 32), b.reshape(n, 32), src2, dst2)
        he = _tc_edge_update(he, ga.reshape(e // 4, 128),
                             gb.reshape(e // 4, 128), lp["edge"],
                             lp["edge_ln"])
        p0, p1 = _sc_scatter(he.reshape(e, 32), dst2, zeros)
        if l + 1 < len(layers):
            hn, a, b = _tc_node_update(hn, p0.reshape(n // 4, 128),
                                       p1.reshape(n // 4, 128), lp["node"],
                                       lp["node_ln"],
                                       ew[l + 1][32:64], ew[l + 1][64:96])
        else:
            hn = _tc_node_update(hn, p0.reshape(n // 4, 128),
                                 p1.reshape(n // 4, 128), lp["node"],
                                 lp["node_ln"])

    return _tc_decoder(hn, m_p, params["dec"]).reshape(n, 3)


# speedup vs baseline: 1.9149x; 1.0808x over previous
import functools

import jax
import jax.numpy as jnp
from jax import lax
from jax.experimental import pallas as pl
from jax.experimental.pallas import tpu as pltpu
from jax.experimental.pallas import tpu_sc as plsc

_EPS = 1e-5
_DTC = 0.01


def _ln(y, g, b):
    mu = jnp.mean(y, axis=-1, keepdims=True)
    var = jnp.mean((y - mu) ** 2, axis=-1, keepdims=True)
    return (y - mu) * lax.rsqrt(var + _EPS) * g + b


def _w(shape):
    return pl.BlockSpec(shape, lambda i: tuple(0 for _ in shape))


def _row(block_rows, cols):
    return pl.BlockSpec((block_rows, cols), lambda i: (i, 0))


def _kron4(w):
    return jnp.kron(jnp.eye(4, dtype=jnp.float32), w)


def _t4(v):
    return jnp.tile(v, 4).reshape(1, -1)


def _mavg():
    return jnp.kron(jnp.eye(4, dtype=jnp.float32),
                    jnp.full((32, 32), 1.0 / 32.0, jnp.float32))


def _pln(y, mavg, g, b):
    mu = jnp.dot(y, mavg, preferred_element_type=jnp.float32)
    d = y - mu
    var = jnp.dot(d * d, mavg, preferred_element_type=jnp.float32)
    return d * lax.rsqrt(var + _EPS) * g + b


def _dot(a, b):
    return jnp.dot(a, b, preferred_element_type=jnp.float32)


def _tc_node_encoder(x_p, mp, lnp, wa, wb):
    r = x_p.shape[0]

    def body(x_ref, w1_ref, b1_ref, w2_ref, b2_ref, g_ref, gb_ref, mavg_ref,
             wa_ref, wb_ref, hn_ref, a_ref, b_ref, m_ref):
        xx = x_ref[...]
        h = jnp.maximum(_dot(xx, w1_ref[...]) + b1_ref[...], 0.0)
        y = _dot(h, w2_ref[...]) + b2_ref[...]
        hn = _pln(y, mavg_ref[...], g_ref[...], gb_ref[...])
        hn_ref[...] = hn
        a_ref[...] = _dot(hn, wa_ref[...])
        b_ref[...] = _dot(hn, wb_ref[...])
        cols = []
        for gidx in range(4):
            z0 = xx[:, 128 * gidx + 1:128 * gidx + 2]
            t1 = xx[:, 128 * gidx + 2:128 * gidx + 3] + _DTC
            mg = (z0 <= t1).astype(jnp.float32)
            cols += [mg, mg, mg]
        m_ref[...] = jnp.concatenate(cols, axis=1)

    return pl.pallas_call(
        body,
        grid=(1,),
        in_specs=[_w((r, 512)), _w((512, 128)), _w((1, 128)), _w((128, 128)),
                  _w((1, 128)), _w((1, 128)), _w((1, 128)), _w((128, 128)),
                  _w((128, 128)), _w((128, 128))],
        out_specs=[_w((r, 128)), _w((r, 128)), _w((r, 128)), _w((r, 12))],
        out_shape=[jax.ShapeDtypeStruct((r, 128), jnp.float32),
                   jax.ShapeDtypeStruct((r, 128), jnp.float32),
                   jax.ShapeDtypeStruct((r, 128), jnp.float32),
                   jax.ShapeDtypeStruct((r, 12), jnp.float32)],
    )(x_p, _kron4(mp["w1"]), _t4(mp["b1"]), _kron4(mp["w2"]), _t4(mp["b2"]),
      _t4(lnp["g"]), _t4(lnp["b"]), _mavg(), _kron4(wa), _kron4(wb))


def _tc_edge_encoder(a0, a1, mp, lnp):
    r = a0.shape[0]
    be = 1000
    eye64 = jnp.eye(64, dtype=jnp.float32)
    r0 = jnp.kron(eye64, mp["w1"][0:1])
    r1 = jnp.kron(eye64, mp["w1"][1:2])
    b1big = jnp.tile(mp["b1"], 64).reshape(1, -1)

    def body(a0_ref, a1_ref, r0_ref, r1_ref, b1_ref, w2_ref, b2_ref,
             g_ref, gb_ref, mavg_ref, out_ref):
        h = jnp.maximum(_dot(a0_ref[...], r0_ref[...])
                        + _dot(a1_ref[...], r1_ref[...]) + b1_ref[...], 0.0)
        ys = []
        for t in range(16):
            y = _dot(h[:, 128 * t:128 * t + 128], w2_ref[...]) + b2_ref[...]
            ys.append(_pln(y, mavg_ref[...], g_ref[...], gb_ref[...]))
        out_ref[...] = jnp.concatenate(ys, axis=1).reshape(16 * be, 128)

    return pl.pallas_call(
        body,
        grid=(r // be,),
        in_specs=[_row(be, 64), _row(be, 64), _w((64, 2048)), _w((64, 2048)),
                  _w((1, 2048)), _w((128, 128)), _w((1, 128)), _w((1, 128)),
                  _w((1, 128)), _w((128, 128))],
        out_specs=_row(16 * be, 128),
        out_shape=jax.ShapeDtypeStruct((16 * r, 128), jnp.float32),
    )(a0, a1, r0, r1, b1big, _kron4(mp["w2"]), _t4(mp["b2"]), _t4(lnp["g"]),
      _t4(lnp["b"]), _mavg())


def _tc_edge_update(he_p, g_p, mp, lnp):
    r = he_p.shape[0]
    be = 2000

    def body(he_ref, gi_ref, w1_ref, b1_ref, w2_ref, b2_ref,
             g_ref, gb2_ref, mavg_ref, out_ref):
        hh = he_ref[...]
        pre = _dot(hh, w1_ref[...]) + gi_ref[...] + b1_ref[...]
        h = jnp.maximum(pre, 0.0)
        y = _dot(h, w2_ref[...]) + b2_ref[...]
        out_ref[...] = hh + _pln(y, mavg_ref[...], g_ref[...], gb2_ref[...])

    return pl.pallas_call(
        body,
        grid=(r // be,),
        in_specs=[_row(be, 128)] * 2 + [_w((128, 128)), _w((1, 128)),
                                        _w((128, 128)), _w((1, 128)),
                                        _w((1, 128)), _w((1, 128)),
                                        _w((128, 128))],
        out_specs=_row(be, 128),
        out_shape=jax.ShapeDtypeStruct((r, 128), jnp.float32),
    )(he_p, g_p, _kron4(mp["w1"][0:32]), _t4(mp["b1"]),
      _kron4(mp["w2"]), _t4(mp["b2"]), _t4(lnp["g"]), _t4(lnp["b"]), _mavg())


def _tc_node_update(hn_p, p0_p, p1_p, mp, lnp, wa=None, wb=None):
    r = hn_p.shape[0]
    emit_ab = wa is not None

    def body(hn_ref, p0_ref, p1_ref, w1a_ref, w1b_ref, b1_ref, w2_ref, b2_ref,
             g_ref, gb_ref, mavg_ref, *rest):
        if emit_ab:
            wa_ref, wb_ref, out_ref, a_ref, b_ref = rest
        else:
            (out_ref,) = rest
        hh = hn_ref[...]
        aggr = p0_ref[...] + p1_ref[...]
        pre = (_dot(hh, w1a_ref[...]) + _dot(aggr, w1b_ref[...]) + b1_ref[...])
        h = jnp.maximum(pre, 0.0)
        y = _dot(h, w2_ref[...]) + b2_ref[...]
        hn_new = hh + _pln(y, mavg_ref[...], g_ref[...], gb_ref[...])
        out_ref[...] = hn_new
        if emit_ab:
            a_ref[...] = _dot(hn_new, wa_ref[...])
            b_ref[...] = _dot(hn_new, wb_ref[...])

    in_specs = [_w((r, 128))] * 3 + [_w((128, 128)), _w((128, 128)),
                                     _w((1, 128)), _w((128, 128)),
                                     _w((1, 128)), _w((1, 128)), _w((1, 128)),
                                     _w((128, 128))]
    args = [hn_p, p0_p, p1_p, _kron4(mp["w1"][0:32]), _kron4(mp["w1"][32:64]),
            _t4(mp["b1"]), _kron4(mp["w2"]), _t4(mp["b2"]),
            _t4(lnp["g"]), _t4(lnp["b"]), _mavg()]
    if emit_ab:
        in_specs += [_w((128, 128)), _w((128, 128))]
        args += [_kron4(wa), _kron4(wb)]
        out_specs = [_w((r, 128))] * 3
        out_shape = [jax.ShapeDtypeStruct((r, 128), jnp.float32)] * 3
    else:
        out_specs = _w((r, 128))
        out_shape = jax.ShapeDtypeStruct((r, 128), jnp.float32)
    return pl.pallas_call(
        body, grid=(1,), in_specs=in_specs, out_specs=out_specs,
        out_shape=out_shape)(*args)


def _tc_decoder(hn_p, m_p, mp):
    r = hn_p.shape[0]

    def body(hn_ref, m_ref, w1_ref, b1_ref, w2_ref, b2_ref, out_ref):
        h = jnp.maximum(_dot(hn_ref[...], w1_ref[...]) + b1_ref[...], 0.0)
        y = _dot(h, w2_ref[...]) + b2_ref[...]
        out_ref[...] = y * m_ref[...]

    return pl.pallas_call(
        body,
        grid=(1,),
        in_specs=[_w((r, 128)), _w((r, 12)), _w((128, 128)), _w((1, 128)),
                  _w((128, 12)), _w((1, 12))],
        out_specs=_w((r, 12)),
        out_shape=jax.ShapeDtypeStruct((r, 12), jnp.float32),
    )(hn_p, m_p, _kron4(mp["w1"]), _t4(mp["b1"]), _kron4(mp["w2"]),
      _t4(mp["b2"]))


@functools.cache
def _mesh():
    return plsc.VectorSubcoreMesh(core_axis_name="c", subcore_axis_name="s")


_NOTILE = pltpu.CompilerParams(use_tc_tiling_on_sc=False)
_CHR = 8
_CH = _CHR * 128


def _sc_gather(a, b, src2, dst2):
    e = src2.shape[0] * 128
    ch = 512
    nch = e // ch

    @functools.partial(
        pl.kernel,
        mesh=_mesh(),
        out_type=jax.ShapeDtypeStruct((e, 32), jnp.float32),
        scratch_types=[pltpu.VMEM((4, 128), jnp.int32),
                       pltpu.VMEM((4, 128), jnp.int32),
                       pltpu.VMEM((4, 128), jnp.int32),
                       pltpu.VMEM((4, 128), jnp.int32),
                       pltpu.VMEM((ch, 32), jnp.float32),
                       pltpu.VMEM((ch, 32), jnp.float32),
                       pltpu.VMEM((ch, 32), jnp.float32),
                       pltpu.VMEM((ch, 32), jnp.float32),
                       pltpu.SemaphoreType.DMA,
                       pltpu.SemaphoreType.DMA],
        compiler_params=_NOTILE,
    )
    def k(a_hbm, b_hbm, s_hbm, d_hbm, g_hbm,
          si0, di0, si1, di1, ba0, bb0, ba1, bb1, sg0, sg1):
        c = lax.axis_index("c")
        s = lax.axis_index("s")
        wid = c * 16 + s
        trips = (nch - wid + 31) // 32

        def fire(ich, si, di, ba, bb, sg):
            chn = wid + ich * 32
            pltpu.sync_copy(s_hbm.at[pl.ds(chn * 4, 4)], si)
            pltpu.sync_copy(d_hbm.at[pl.ds(chn * 4, 4)], di)
            for j in range(4):
                pltpu.async_copy(a_hbm.at[si.at[j]],
                                 ba.at[pl.ds(j * 128, 128)], sg)
                pltpu.async_copy(b_hbm.at[di.at[j]],
                                 bb.at[pl.ds(j * 128, 128)], sg)

        def drain(ich, si, di, ba, bb, sg):
            for j in range(4):
                pltpu.make_async_copy(a_hbm.at[si.at[j]],
                                      ba.at[pl.ds(j * 128, 128)], sg).wait()
                pltpu.make_async_copy(b_hbm.at[di.at[j]],
                                      bb.at[pl.ds(j * 128, 128)], sg).wait()

            def arow(i8, carry):
                for u in range(8):
                    r = i8 * 8 + u
                    for hh in range(2):
                        sl = pl.ds(16 * hh, 16)
                        ba[r, sl] = ba[r, sl] + bb[r, sl]
                return carry

            lax.fori_loop(0, ch // 8, arow, 0)
            chn = wid + ich * 32
            pltpu.sync_copy(ba, g_hbm.at[pl.ds(chn * ch, ch)])

        fire(0, si0, di0, ba0, bb0, sg0)

        def body(it, carry):
            i0 = it * 2

            @pl.when(i0 + 1 < trips)
            def _():
                fire(i0 + 1, si1, di1, ba1, bb1, sg1)

            drain(i0, si0, di0, ba0, bb0, sg0)

            @pl.when(i0 + 2 < trips)
            def _():
                fire(i0 + 2, si0, di0, ba0, bb0, sg0)

            @pl.when(i0 + 1 < trips)
            def _():
                drain(i0 + 1, si1, di1, ba1, bb1, sg1)

            return carry

        lax.fori_loop(0, (trips + 1) // 2, body, 0)

    return k(a, b, src2, dst2)


def _sc_scatter(he, dst2, zeros):
    n = zeros.shape[0]
    e = he.shape[0]
    nch = e // _CH
    tail = (e - nch * _CH) // 128
    per = n // 16

    @functools.partial(
        pl.kernel,
        mesh=_mesh(),
        out_type=[jax.ShapeDtypeStruct((n, 32), jnp.float32),
                  jax.ShapeDtypeStruct((n, 32), jnp.float32)],
        scratch_types=[pltpu.VMEM((_CHR, 128), jnp.int32),
                       pltpu.VMEM((_CH, 32), jnp.float32),
                       pltpu.VMEM_SHARED((n, 32), jnp.float32)],
        compiler_params=_NOTILE,
    )
    def k(he_hbm, d_hbm, z_hbm, o0, o1, di, be, acc):
        c = lax.axis_index("c")
        s = lax.axis_index("s")
        wid = c * 16 + s
        pltpu.sync_copy(z_hbm.at[pl.ds(s * per, per)], acc.at[pl.ds(s * per, per)])
        plsc.subcore_barrier()
        trips = (nch - wid + 31) // 32

        def do_chunk(ch, rows):
            pltpu.sync_copy(d_hbm.at[pl.ds(ch * _CHR, rows)],
                            di.at[pl.ds(0, rows)])
            pltpu.sync_copy(he_hbm.at[pl.ds(ch * _CH, rows * 128)],
                            be.at[pl.ds(0, rows * 128)])
            for j in range(rows):
                pltpu.sync_copy(be.at[pl.ds(j * 128, 128)], acc.at[di.at[j]],
                                add=True)

        def body(i, carry):
            do_chunk(wid + i * 32, _CHR)
            return carry

        lax.fori_loop(0, trips, body, 0)
        if tail:
            @pl.when(wid == 30)
            def _():
                do_chunk(nch, tail)
        plsc.subcore_barrier()

        @pl.when(c == 0)
        def _():
            pltpu.sync_copy(acc.at[pl.ds(s * per, per)], o0.at[pl.ds(s * per, per)])

        @pl.when(c == 1)
        def _():
            pltpu.sync_copy(acc.at[pl.ds(s * per, per)], o1.at[pl.ds(s * per, per)])

    return k(he, dst2, zeros)


def kernel(x, edge_attr, params, edge_index):
    n = x.shape[0]
    e = edge_index.shape[1]
    src2 = edge_index[0].reshape(-1, 128)
    dst2 = edge_index[1].reshape(-1, 128)
    layers = params["layers"]
    ew = [lp["edge"]["w1"] for lp in layers]

    hn, a, b, m_p = _tc_node_encoder(x.reshape(n // 4, 512), params["enc_n"],
                                     params["enc_n_ln"],
                                     ew[0][32:64], ew[0][64:96])
    he = _tc_edge_encoder(edge_attr[:, 0].reshape(e // 64, 64),
                          edge_attr[:, 1].reshape(e // 64, 64),
                          params["enc_e"], params["enc_e_ln"])
    zeros = jnp.zeros((n, 32), jnp.float32)

    for l, lp in enumerate(layers):
        g = _sc_gather(a.reshape(n, 32), b.reshape(n, 32), src2, dst2)
        he = _tc_edge_update(he, g.reshape(e // 4, 128), lp["edge"],
                             lp["edge_ln"])
        p0, p1 = _sc_scatter(he.reshape(e, 32), dst2, zeros)
        if l + 1 < len(layers):
            hn, a, b = _tc_node_update(hn, p0.reshape(n // 4, 128),
                                       p1.reshape(n // 4, 128), lp["node"],
                                       lp["node_ln"],
                                       ew[l + 1][32:64], ew[l + 1][64:96])
        else:
            hn = _tc_node_update(hn, p0.reshape(n // 4, 128),
                                 p1.reshape(n // 4, 128), lp["node"],
                                 lp["node_ln"])

    return _tc_decoder(hn, m_p, params["dec"]).reshape(n, 3)


# speedup vs baseline: 2.1206x; 1.1074x over previous
import functools

import jax
import jax.numpy as jnp
from jax import lax
from jax.experimental import pallas as pl
from jax.experimental.pallas import tpu as pltpu
from jax.experimental.pallas import tpu_sc as plsc

_EPS = 1e-5
_DTC = 0.01


def _ln(y, g, b):
    mu = jnp.mean(y, axis=-1, keepdims=True)
    var = jnp.mean((y - mu) ** 2, axis=-1, keepdims=True)
    return (y - mu) * lax.rsqrt(var + _EPS) * g + b


def _w(shape):
    return pl.BlockSpec(shape, lambda i: tuple(0 for _ in shape))


def _row(block_rows, cols):
    return pl.BlockSpec((block_rows, cols), lambda i: (i, 0))


def _kron4(w):
    return jnp.kron(jnp.eye(4, dtype=jnp.float32), w)


def _t4(v):
    return jnp.tile(v, 4).reshape(1, -1)


def _mavg():
    return jnp.kron(jnp.eye(4, dtype=jnp.float32),
                    jnp.full((32, 32), 1.0 / 32.0, jnp.float32))


def _pln(y, mavg, g, b):
    mu = jnp.dot(y, mavg, preferred_element_type=jnp.float32)
    d = y - mu
    var = jnp.dot(d * d, mavg, preferred_element_type=jnp.float32)
    return d * lax.rsqrt(var + _EPS) * g + b


def _dot(a, b):
    return jnp.dot(a, b, preferred_element_type=jnp.float32)


def _tc_node_encoder(x_p, mp, lnp, wa, wb):
    r = x_p.shape[0]

    def body(x_ref, w1_ref, b1_ref, w2_ref, b2_ref, g_ref, gb_ref, mavg_ref,
             wa_ref, wb_ref, hn_ref, a_ref, b_ref, m_ref):
        xx = x_ref[...]
        h = jnp.maximum(_dot(xx, w1_ref[...]) + b1_ref[...], 0.0)
        y = _dot(h, w2_ref[...]) + b2_ref[...]
        hn = _pln(y, mavg_ref[...], g_ref[...], gb_ref[...])
        hn_ref[...] = hn
        a_ref[...] = _dot(hn, wa_ref[...])
        b_ref[...] = _dot(hn, wb_ref[...])
        cols = []
        for gidx in range(4):
            z0 = xx[:, 128 * gidx + 1:128 * gidx + 2]
            t1 = xx[:, 128 * gidx + 2:128 * gidx + 3] + _DTC
            mg = (z0 <= t1).astype(jnp.float32)
            cols += [mg, mg, mg]
        m_ref[...] = jnp.concatenate(cols, axis=1)

    return pl.pallas_call(
        body,
        grid=(1,),
        in_specs=[_w((r, 512)), _w((512, 128)), _w((1, 128)), _w((128, 128)),
                  _w((1, 128)), _w((1, 128)), _w((1, 128)), _w((128, 128)),
                  _w((128, 128)), _w((128, 128))],
        out_specs=[_w((r, 128)), _w((r, 128)), _w((r, 128)), _w((r, 12))],
        out_shape=[jax.ShapeDtypeStruct((r, 128), jnp.float32),
                   jax.ShapeDtypeStruct((r, 128), jnp.float32),
                   jax.ShapeDtypeStruct((r, 128), jnp.float32),
                   jax.ShapeDtypeStruct((r, 12), jnp.float32)],
    )(x_p, _kron4(mp["w1"]), _t4(mp["b1"]), _kron4(mp["w2"]), _t4(mp["b2"]),
      _t4(lnp["g"]), _t4(lnp["b"]), _mavg(), _kron4(wa), _kron4(wb))


def _tc_edge_encoder(a0, a1, mp, lnp):
    r = a0.shape[0]
    be = 1000
    eye64 = jnp.eye(64, dtype=jnp.float32)
    r0 = jnp.kron(eye64, mp["w1"][0:1])
    r1 = jnp.kron(eye64, mp["w1"][1:2])
    b1big = jnp.tile(mp["b1"], 64).reshape(1, -1)

    def body(a0_ref, a1_ref, r0_ref, r1_ref, b1_ref, w2_ref, b2_ref,
             g_ref, gb_ref, mavg_ref, out_ref):
        h = jnp.maximum(_dot(a0_ref[...], r0_ref[...])
                        + _dot(a1_ref[...], r1_ref[...]) + b1_ref[...], 0.0)
        ys = []
        for t in range(16):
            y = _dot(h[:, 128 * t:128 * t + 128], w2_ref[...]) + b2_ref[...]
            ys.append(_pln(y, mavg_ref[...], g_ref[...], gb_ref[...]))
        out_ref[...] = jnp.concatenate(ys, axis=1).reshape(16 * be, 128)

    return pl.pallas_call(
        body,
        grid=(r // be,),
        in_specs=[_row(be, 64), _row(be, 64), _w((64, 2048)), _w((64, 2048)),
                  _w((1, 2048)), _w((128, 128)), _w((1, 128)), _w((1, 128)),
                  _w((1, 128)), _w((128, 128))],
        out_specs=_row(16 * be, 128),
        out_shape=jax.ShapeDtypeStruct((16 * r, 128), jnp.float32),
    )(a0, a1, r0, r1, b1big, _kron4(mp["w2"]), _t4(mp["b2"]), _t4(lnp["g"]),
      _t4(lnp["b"]), _mavg())


def _tc_edge_update(he_p, g_p, mp, lnp):
    r = he_p.shape[0]
    be = 2000

    def body(he_ref, gi_ref, w1_ref, b1_ref, w2_ref, b2_ref,
             g_ref, gb2_ref, mavg_ref, out_ref):
        hh = he_ref[...]
        pre = _dot(hh, w1_ref[...]) + gi_ref[...] + b1_ref[...]
        h = jnp.maximum(pre, 0.0)
        y = _dot(h, w2_ref[...]) + b2_ref[...]
        out_ref[...] = hh + _pln(y, mavg_ref[...], g_ref[...], gb2_ref[...])

    return pl.pallas_call(
        body,
        grid=(r // be,),
        in_specs=[_row(be, 128)] * 2 + [_w((128, 128)), _w((1, 128)),
                                        _w((128, 128)), _w((1, 128)),
                                        _w((1, 128)), _w((1, 128)),
                                        _w((128, 128))],
        out_specs=_row(be, 128),
        out_shape=jax.ShapeDtypeStruct((r, 128), jnp.float32),
    )(he_p, g_p, _kron4(mp["w1"][0:32]), _t4(mp["b1"]),
      _kron4(mp["w2"]), _t4(mp["b2"]), _t4(lnp["g"]), _t4(lnp["b"]), _mavg())


def _tc_node_update(hn_p, p0_p, p1_p, mp, lnp, wa=None, wb=None):
    r = hn_p.shape[0]
    emit_ab = wa is not None

    def body(hn_ref, p0_ref, p1_ref, w1a_ref, w1b_ref, b1_ref, w2_ref, b2_ref,
             g_ref, gb_ref, mavg_ref, *rest):
        if emit_ab:
            wa_ref, wb_ref, out_ref, a_ref, b_ref = rest
        else:
            (out_ref,) = rest
        hh = hn_ref[...]
        aggr = p0_ref[...] + p1_ref[...]
        pre = (_dot(hh, w1a_ref[...]) + _dot(aggr, w1b_ref[...]) + b1_ref[...])
        h = jnp.maximum(pre, 0.0)
        y = _dot(h, w2_ref[...]) + b2_ref[...]
        hn_new = hh + _pln(y, mavg_ref[...], g_ref[...], gb_ref[...])
        out_ref[...] = hn_new
        if emit_ab:
            a_ref[...] = _dot(hn_new, wa_ref[...])
            b_ref[...] = _dot(hn_new, wb_ref[...])

    in_specs = [_w((r, 128))] * 3 + [_w((128, 128)), _w((128, 128)),
                                     _w((1, 128)), _w((128, 128)),
                                     _w((1, 128)), _w((1, 128)), _w((1, 128)),
                                     _w((128, 128))]
    args = [hn_p, p0_p, p1_p, _kron4(mp["w1"][0:32]), _kron4(mp["w1"][32:64]),
            _t4(mp["b1"]), _kron4(mp["w2"]), _t4(mp["b2"]),
            _t4(lnp["g"]), _t4(lnp["b"]), _mavg()]
    if emit_ab:
        in_specs += [_w((128, 128)), _w((128, 128))]
        args += [_kron4(wa), _kron4(wb)]
        out_specs = [_w((r, 128))] * 3
        out_shape = [jax.ShapeDtypeStruct((r, 128), jnp.float32)] * 3
    else:
        out_specs = _w((r, 128))
        out_shape = jax.ShapeDtypeStruct((r, 128), jnp.float32)
    return pl.pallas_call(
        body, grid=(1,), in_specs=in_specs, out_specs=out_specs,
        out_shape=out_shape)(*args)


def _tc_decoder(hn_p, m_p, mp):
    r = hn_p.shape[0]

    def body(hn_ref, m_ref, w1_ref, b1_ref, w2_ref, b2_ref, out_ref):
        h = jnp.maximum(_dot(hn_ref[...], w1_ref[...]) + b1_ref[...], 0.0)
        y = _dot(h, w2_ref[...]) + b2_ref[...]
        out_ref[...] = y * m_ref[...]

    return pl.pallas_call(
        body,
        grid=(1,),
        in_specs=[_w((r, 128)), _w((r, 12)), _w((128, 128)), _w((1, 128)),
                  _w((128, 12)), _w((1, 12))],
        out_specs=_w((r, 12)),
        out_shape=jax.ShapeDtypeStruct((r, 12), jnp.float32),
    )(hn_p, m_p, _kron4(mp["w1"]), _t4(mp["b1"]), _kron4(mp["w2"]),
      _t4(mp["b2"]))


@functools.cache
def _mesh():
    return plsc.VectorSubcoreMesh(core_axis_name="c", subcore_axis_name="s")


_NOTILE = pltpu.CompilerParams(use_tc_tiling_on_sc=False)
_CHR = 8
_CH = _CHR * 128


def _sc_gather(a, b, src2, dst2):
    e = src2.shape[0] * 128
    ch = 512
    nch = e // ch

    @functools.partial(
        pl.kernel,
        mesh=_mesh(),
        out_type=jax.ShapeDtypeStruct((e, 32), jnp.float32),
        scratch_types=[pltpu.VMEM((4, 128), jnp.int32),
                       pltpu.VMEM((4, 128), jnp.int32),
                       pltpu.VMEM((4, 128), jnp.int32),
                       pltpu.VMEM((4, 128), jnp.int32),
                       pltpu.VMEM((ch, 32), jnp.float32),
                       pltpu.VMEM((ch, 32), jnp.float32),
                       pltpu.VMEM((ch, 32), jnp.float32),
                       pltpu.VMEM((ch, 32), jnp.float32),
                       pltpu.VMEM_SHARED((a.shape[0], 32), jnp.float32),
                       pltpu.VMEM_SHARED((a.shape[0], 32), jnp.float32),
                       pltpu.SemaphoreType.DMA,
                       pltpu.SemaphoreType.DMA],
        compiler_params=_NOTILE,
    )
    def k(a_hbm, b_hbm, s_hbm, d_hbm, g_hbm,
          si0, di0, si1, di1, ba0, bb0, ba1, bb1, ash, bsh, sg0, sg1):
        c = lax.axis_index("c")
        s = lax.axis_index("s")
        wid = c * 16 + s
        trips = (nch - wid + 31) // 32
        nper = a_hbm.shape[0] // 16
        pltpu.sync_copy(a_hbm.at[pl.ds(s * nper, nper)],
                        ash.at[pl.ds(s * nper, nper)])
        pltpu.sync_copy(b_hbm.at[pl.ds(s * nper, nper)],
                        bsh.at[pl.ds(s * nper, nper)])
        plsc.subcore_barrier()

        def fire(ich, si, di, ba, bb, sg):
            chn = wid + ich * 32
            pltpu.sync_copy(s_hbm.at[pl.ds(chn * 4, 4)], si)
            pltpu.sync_copy(d_hbm.at[pl.ds(chn * 4, 4)], di)
            for j in range(4):
                pltpu.async_copy(ash.at[si.at[j]],
                                 ba.at[pl.ds(j * 128, 128)], sg)
                pltpu.async_copy(bsh.at[di.at[j]],
                                 bb.at[pl.ds(j * 128, 128)], sg)

        def drain(ich, si, di, ba, bb, sg):
            for j in range(4):
                pltpu.make_async_copy(ash.at[si.at[j]],
                                      ba.at[pl.ds(j * 128, 128)], sg).wait()
                pltpu.make_async_copy(bsh.at[di.at[j]],
                                      bb.at[pl.ds(j * 128, 128)], sg).wait()

            def arow(i8, carry):
                for u in range(8):
                    r = i8 * 8 + u
                    for hh in range(2):
                        sl = pl.ds(16 * hh, 16)
                        ba[r, sl] = ba[r, sl] + bb[r, sl]
                return carry

            lax.fori_loop(0, ch // 8, arow, 0)
            chn = wid + ich * 32
            pltpu.sync_copy(ba, g_hbm.at[pl.ds(chn * ch, ch)])

        fire(0, si0, di0, ba0, bb0, sg0)

        def body(it, carry):
            i0 = it * 2

            @pl.when(i0 + 1 < trips)
            def _():
                fire(i0 + 1, si1, di1, ba1, bb1, sg1)

            drain(i0, si0, di0, ba0, bb0, sg0)

            @pl.when(i0 + 2 < trips)
            def _():
                fire(i0 + 2, si0, di0, ba0, bb0, sg0)

            @pl.when(i0 + 1 < trips)
            def _():
                drain(i0 + 1, si1, di1, ba1, bb1, sg1)

            return carry

        lax.fori_loop(0, (trips + 1) // 2, body, 0)

    return k(a, b, src2, dst2)


def _sc_scatter(he, dst2, zeros):
    n = zeros.shape[0]
    e = he.shape[0]
    nch = e // _CH
    tail = (e - nch * _CH) // 128
    per = n // 16

    @functools.partial(
        pl.kernel,
        mesh=_mesh(),
        out_type=[jax.ShapeDtypeStruct((n, 32), jnp.float32),
                  jax.ShapeDtypeStruct((n, 32), jnp.float32)],
        scratch_types=[pltpu.VMEM((_CHR, 128), jnp.int32),
                       pltpu.VMEM((_CHR, 128), jnp.int32),
                       pltpu.VMEM((_CH, 32), jnp.float32),
                       pltpu.VMEM((_CH, 32), jnp.float32),
                       pltpu.VMEM_SHARED((n, 32), jnp.float32),
                       pltpu.SemaphoreType.DMA,
                       pltpu.SemaphoreType.DMA,
                       pltpu.SemaphoreType.DMA],
        compiler_params=_NOTILE,
    )
    def k(he_hbm, d_hbm, z_hbm, o0, o1, di0, di1, be0, be1, acc,
          sl0, sl1, ss):
        c = lax.axis_index("c")
        s = lax.axis_index("s")
        wid = c * 16 + s
        pltpu.sync_copy(z_hbm.at[pl.ds(s * per, per)], acc.at[pl.ds(s * per, per)])
        plsc.subcore_barrier()
        trips = (nch - wid + 31) // 32

        def load(i, di, be, sl):
            chn = wid + i * 32
            pltpu.async_copy(d_hbm.at[pl.ds(chn * _CHR, _CHR)], di, sl)
            pltpu.async_copy(he_hbm.at[pl.ds(chn * _CH, _CH)], be, sl)

        def proc(i, di, be, sl):
            chn = wid + i * 32
            pltpu.make_async_copy(d_hbm.at[pl.ds(chn * _CHR, _CHR)],
                                  di, sl).wait()
            pltpu.make_async_copy(he_hbm.at[pl.ds(chn * _CH, _CH)],
                                  be, sl).wait()
            cps = [pltpu.async_copy(be.at[pl.ds(j * 128, 128)],
                                    acc.at[di.at[j]], ss, add=True)
                   for j in range(_CHR)]
            for cp in cps:
                cp.wait()

        load(0, di0, be0, sl0)

        def body(it, carry):
            i0 = it * 2

            @pl.when(i0 + 1 < trips)
            def _():
                load(i0 + 1, di1, be1, sl1)

            proc(i0, di0, be0, sl0)

            @pl.when(i0 + 2 < trips)
            def _():
                load(i0 + 2, di0, be0, sl0)

            @pl.when(i0 + 1 < trips)
            def _():
                proc(i0 + 1, di1, be1, sl1)

            return carry

        lax.fori_loop(0, (trips + 1) // 2, body, 0)

        if tail:
            @pl.when(wid == 30)
            def _():
                pltpu.sync_copy(d_hbm.at[pl.ds(nch * _CHR, tail)],
                                di0.at[pl.ds(0, tail)])
                pltpu.sync_copy(he_hbm.at[pl.ds(nch * _CH, tail * 128)],
                                be0.at[pl.ds(0, tail * 128)])
                for j in range(tail):
                    pltpu.sync_copy(be0.at[pl.ds(j * 128, 128)],
                                    acc.at[di0.at[j]], add=True)
        plsc.subcore_barrier()

        @pl.when(c == 0)
        def _():
            pltpu.sync_copy(acc.at[pl.ds(s * per, per)], o0.at[pl.ds(s * per, per)])

        @pl.when(c == 1)
        def _():
            pltpu.sync_copy(acc.at[pl.ds(s * per, per)], o1.at[pl.ds(s * per, per)])

    return k(he, dst2, zeros)


def kernel(x, edge_attr, params, edge_index):
    n = x.shape[0]
    e = edge_index.shape[1]
    src2 = edge_index[0].reshape(-1, 128)
    dst2 = edge_index[1].reshape(-1, 128)
    layers = params["layers"]
    ew = [lp["edge"]["w1"] for lp in layers]

    hn, a, b, m_p = _tc_node_encoder(x.reshape(n // 4, 512), params["enc_n"],
                                     params["enc_n_ln"],
                                     ew[0][32:64], ew[0][64:96])
    he = _tc_edge_encoder(edge_attr[:, 0].reshape(e // 64, 64),
                          edge_attr[:, 1].reshape(e // 64, 64),
                          params["enc_e"], params["enc_e_ln"])
    zeros = jnp.zeros((n, 32), jnp.float32)

    for l, lp in enumerate(layers):
        g = _sc_gather(a.reshape(n, 32), b.reshape(n, 32), src2, dst2)
        he = _tc_edge_update(he, g.reshape(e // 4, 128), lp["edge"],
                             lp["edge_ln"])
        p0, p1 = _sc_scatter(he.reshape(e, 32), dst2, zeros)
        if l + 1 < len(layers):
            hn, a, b = _tc_node_update(hn, p0.reshape(n // 4, 128),
                                       p1.reshape(n // 4, 128), lp["node"],
                                       lp["node_ln"],
                                       ew[l + 1][32:64], ew[l + 1][64:96])
        else:
            hn = _tc_node_update(hn, p0.reshape(n // 4, 128),
                                 p1.reshape(n // 4, 128), lp["node"],
                                 lp["node_ln"])

    return _tc_decoder(hn, m_p, params["dec"]).reshape(n, 3)
